# baseline (speedup 1.0000x reference)
import functools
import jax
import jax.numpy as jnp
from jax import lax
from jax.experimental import pallas as pl
from jax.experimental.pallas import tpu as pltpu
from jax.experimental.pallas import tpu_sc as plsc

V2, K, D = 100000, 1000, 64
RB = 2000
NBLK = V2 // RB
NBUF = 4

BATCH, SEQ = 4096, 50
B = BATCH * SEQ
NC, NS = 2, 16
NW = NC * NS
BPW = B // NW
CHUNK = 128
NCHUNK = BPW // CHUNK


def _table_body(m_hbm, l1_ref, out_ref, bufs, sems):
    i = pl.program_id(0)

    def start(blk, slot):
        pltpu.make_async_copy(
            m_hbm.at[pl.ds(blk * RB, RB), :], bufs.at[slot], sems.at[slot]
        ).start()

    @pl.when(i == 0)
    def _():
        for b in range(NBUF):
            start(b, b)

    @pl.when((i > 0) & (i + NBUF - 1 < NBLK))
    def _():
        start(i + NBUF - 1, (i + NBUF - 1) % NBUF)

    l1 = l1_ref[...]
    for b in range(NBUF):
        @pl.when(i % NBUF == b)
        def _(b=b):
            pltpu.make_async_copy(
                m_hbm.at[pl.ds(0, RB), :], bufs.at[b], sems.at[b]
            ).wait()
            m = bufs[b]
            mx = jnp.max(m, axis=1, keepdims=True)
            e = jnp.exp(m - mx)
            s = jnp.sum(e, axis=1, keepdims=True)
            out_ref[...] = jnp.dot(e, l1, preferred_element_type=jnp.float32) / s


def _build_table(map_weights, l1_weights):
    return pl.pallas_call(
        _table_body,
        grid=(NBLK,),
        in_specs=[
            pl.BlockSpec(memory_space=pl.ANY),
            pl.BlockSpec((K, D), lambda i: (0, 0)),
        ],
        out_specs=pl.BlockSpec((RB, D), lambda i: (i, 0)),
        out_shape=jax.ShapeDtypeStruct((V2, D), jnp.float32),
        scratch_shapes=[
            pltpu.VMEM((NBUF, RB, K), jnp.float32),
            pltpu.SemaphoreType.DMA((NBUF,)),
        ],
    )(map_weights, l1_weights)


def _gather_body(table_hbm, x_hbm, out_hbm, idx_v, rows_v, gsem, wsem):
    wid = lax.axis_index("s") * NC + lax.axis_index("c")
    pltpu.sync_copy(x_hbm.at[wid], idx_v)
    base = wid * BPW

    def g_copy(j, slot):
        return pltpu.make_async_copy(
            table_hbm.at[idx_v.at[j]], rows_v.at[slot], gsem.at[slot])

    def w_copy(j, slot):
        return pltpu.make_async_copy(
            rows_v.at[slot], out_hbm.at[pl.ds(base + j * CHUNK, CHUNK)],
            wsem.at[slot])

    for p in range(7):
        g_copy(p, p).start()

    def body(j, carry):
        slot = j % 8
        nslot = (j + 7) % 8

        @pl.when(j + 7 < NCHUNK)
        def _():
            @pl.when(j >= 1)
            def _():
                w_copy(j - 1, nslot).wait()
            g_copy(j + 7, nslot).start()

        g_copy(j, slot).wait()
        w_copy(j, slot).start()
        return carry

    lax.fori_loop(0, NCHUNK, body, 0)
    for p in range(8):
        w_copy(NCHUNK - 8 + p, (NCHUNK - 8 + p) % 8).wait()


_gather = functools.partial(
    pl.kernel,
    mesh=plsc.VectorSubcoreMesh(core_axis_name="c", subcore_axis_name="s"),
    out_type=jax.ShapeDtypeStruct((B, D), jnp.float32),
    scratch_types=[
        pltpu.VMEM((NCHUNK, CHUNK), jnp.int32),
        pltpu.VMEM((8, CHUNK, D), jnp.float32),
        pltpu.SemaphoreType.DMA((8,)),
        pltpu.SemaphoreType.DMA((8,)),
    ],
    compiler_params=pltpu.CompilerParams(use_tc_tiling_on_sc=False),
)(_gather_body)


def kernel(x, l1_weights, map_weights):
    table = _build_table(map_weights, l1_weights)
    idx = x.reshape(NW, NCHUNK, CHUNK).astype(jnp.int32)
    out = _gather(table, idx)
    return out.reshape(x.shape[0], x.shape[1], D)


# speedup vs baseline: 1.0013x; 1.0013x over previous
import functools
import jax
import jax.numpy as jnp
from jax import lax
from jax.experimental import pallas as pl
from jax.experimental.pallas import tpu as pltpu
from jax.experimental.pallas import tpu_sc as plsc

V2, K, D = 100000, 1000, 64
RB = 2000
NBLK = V2 // RB
NBUF = 4

BATCH, SEQ = 4096, 50
B = BATCH * SEQ
NC, NS = 2, 16
NW = NC * NS
BPW = B // NW
CHUNK = 128
NCHUNK = BPW // CHUNK


def _table_body(m_hbm, l1_ref, out_ref, bufs, sems):
    i = pl.program_id(0)

    def start(blk, slot):
        pltpu.make_async_copy(
            m_hbm.at[pl.ds(blk * RB, RB), :], bufs.at[slot], sems.at[slot]
        ).start()

    @pl.when(i == 0)
    def _():
        for b in range(NBUF):
            start(b, b)

    @pl.when((i > 0) & (i + NBUF - 1 < NBLK))
    def _():
        start(i + NBUF - 1, (i + NBUF - 1) % NBUF)

    l1 = l1_ref[...]
    for b in range(NBUF):
        @pl.when(i % NBUF == b)
        def _(b=b):
            pltpu.make_async_copy(
                m_hbm.at[pl.ds(0, RB), :], bufs.at[b], sems.at[b]
            ).wait()
            m = bufs[b]
            mx = jnp.max(m, axis=1, keepdims=True)
            e = jnp.exp(m - mx)
            s = jnp.sum(e, axis=1, keepdims=True)
            out_ref[...] = jnp.dot(e, l1, preferred_element_type=jnp.float32) / s


def _build_table(map_weights, l1_weights):
    return pl.pallas_call(
        _table_body,
        grid=(NBLK,),
        in_specs=[
            pl.BlockSpec(memory_space=pl.ANY),
            pl.BlockSpec((K, D), lambda i: (0, 0)),
        ],
        out_specs=pl.BlockSpec((RB, D), lambda i: (i, 0)),
        out_shape=jax.ShapeDtypeStruct((V2, D), jnp.float32),
        scratch_shapes=[
            pltpu.VMEM((NBUF, RB, K), jnp.float32),
            pltpu.SemaphoreType.DMA((NBUF,)),
        ],
    )(map_weights, l1_weights)


def _gather_body(table_hbm, x_hbm, out_hbm, idx_v, rows_v, gsem, wsem):
    wid = lax.axis_index("s") * NC + lax.axis_index("c")
    pltpu.sync_copy(x_hbm.at[wid], idx_v)
    base = wid * BPW

    def g_copy(j, slot):
        return pltpu.make_async_copy(
            table_hbm.at[idx_v.at[j]], rows_v.at[slot], gsem.at[slot])

    def w_copy(j, slot):
        return pltpu.make_async_copy(
            rows_v.at[slot], out_hbm.at[pl.ds(base + j * CHUNK, CHUNK)],
            wsem.at[slot])

    for p in range(3):
        g_copy(p, p).start()

    def body(j, carry):
        slot = j % 4
        nslot = (j + 3) % 4

        @pl.when(j + 3 < NCHUNK)
        def _():
            @pl.when(j >= 1)
            def _():
                w_copy(j - 1, nslot).wait()
            g_copy(j + 3, nslot).start()

        g_copy(j, slot).wait()
        w_copy(j, slot).start()
        return carry

    lax.fori_loop(0, NCHUNK, body, 0)
    for p in range(4):
        w_copy(NCHUNK - 4 + p, (NCHUNK - 4 + p) % 4).wait()


_gather = functools.partial(
    pl.kernel,
    mesh=plsc.VectorSubcoreMesh(core_axis_name="c", subcore_axis_name="s"),
    out_type=jax.ShapeDtypeStruct((B, D), jnp.float32),
    scratch_types=[
        pltpu.VMEM((NCHUNK, CHUNK), jnp.int32),
        pltpu.VMEM((4, CHUNK, D), jnp.float32),
        pltpu.SemaphoreType.DMA((4,)),
        pltpu.SemaphoreType.DMA((4,)),
    ],
    compiler_params=pltpu.CompilerParams(use_tc_tiling_on_sc=False),
)(_gather_body)


def kernel(x, l1_weights, map_weights):
    table = _build_table(map_weights, l1_weights)
    idx = x.reshape(NW, NCHUNK, CHUNK).astype(jnp.int32)
    out = _gather(table, idx)
    return out.reshape(x.shape[0], x.shape[1], D)
